# baseline (speedup 1.0000x reference)
import functools

import jax
import jax.numpy as jnp
from jax import lax
from jax.experimental import pallas as pl
from jax.experimental.pallas import tpu as pltpu
from jax.experimental.pallas import tpu_sc as plsc

B = 16384
K = 32
NC = 2
NS = 16
NW = NC * NS
BPW = B // NW
GROUPS = BPW // 16


def _mfreg_body(u_hbm, b_hbm, uvec_hbm, bvec_hbm, ub_hbm, bb_hbm,
                out_hbm, u_idx, b_idx, u_rows, b_rows, ub_rows, bb_rows,
                out_v, sem):
    c = lax.axis_index("c")
    s = lax.axis_index("s")
    wid = s * NC + c
    base = pl.multiple_of(wid * BPW, BPW)

    pltpu.sync_copy(u_hbm.at[pl.ds(base, BPW)], u_idx)
    pltpu.sync_copy(b_hbm.at[pl.ds(base, BPW)], b_idx)
    cp1 = pltpu.async_copy(uvec_hbm.at[u_idx], u_rows, sem)
    cp2 = pltpu.async_copy(bvec_hbm.at[b_idx], b_rows, sem)
    cp3 = pltpu.async_copy(ub_hbm.at[u_idx], ub_rows, sem)
    cp4 = pltpu.async_copy(bb_hbm.at[b_idx], bb_rows, sem)
    cp1.wait()
    cp2.wait()
    cp3.wait()
    cp4.wait()

    lanes = lax.iota(jnp.int32, 16)

    def group_body(g, carry):
        rows = g * 16 + lanes
        off = pl.multiple_of(g * 16, 16)
        acc = ub_rows[pl.ds(off, 16)] + bb_rows[pl.ds(off, 16)]
        for k in range(K):
            kk = jnp.full((16,), k, jnp.int32)
            acc += (plsc.load_gather(u_rows, [rows, kk])
                    * plsc.load_gather(b_rows, [rows, kk]))
        out_v[pl.ds(pl.multiple_of(g * 16, 16), 16)] = acc
        return carry

    lax.fori_loop(0, GROUPS, group_body, 0)
    pltpu.sync_copy(out_v, out_hbm.at[pl.ds(base, BPW)])


@jax.jit
def _mfreg(u, b, u_vec, b_vec, u_b, b_b, mu):
    mesh = plsc.VectorSubcoreMesh(core_axis_name="c", subcore_axis_name="s")
    dots = pl.kernel(
        _mfreg_body,
        out_type=jax.ShapeDtypeStruct((B,), jnp.float32),
        mesh=mesh,
        compiler_params=pltpu.CompilerParams(
            needs_layout_passes=False, use_tc_tiling_on_sc=False),
        scratch_types=[
            pltpu.VMEM((BPW,), jnp.int32),
            pltpu.VMEM((BPW,), jnp.int32),
            pltpu.VMEM((BPW, K), jnp.float32),
            pltpu.VMEM((BPW, K), jnp.float32),
            pltpu.VMEM((BPW,), jnp.float32),
            pltpu.VMEM((BPW,), jnp.float32),
            pltpu.VMEM((BPW,), jnp.float32),
            pltpu.SemaphoreType.DMA,
        ],
    )(u, b, u_vec, b_vec, jnp.squeeze(u_b, 1), jnp.squeeze(b_b, 1))
    return dots + mu[0]


def kernel(u, b, u_vec, b_vec, u_b, b_b, mu):
    return _mfreg(u, b, u_vec, b_vec, u_b, b_b, mu)
